# baseline (speedup 1.0000x reference)
import functools

import jax
import jax.numpy as jnp
from jax import lax
from jax.experimental import pallas as pl
from jax.experimental.pallas import tpu as pltpu
from jax.experimental.pallas import tpu_sc as plsc

_NUM_NODES = 10000
_B = 1_000_000
_B_PAD = 1 << 20
_NC, _NS, _L = 2, 16, 16
_NW = _NC * _NS
_PW = _B_PAD // _NW
_C = 2048
_NCH = _PW // _C
_IN_ITERS = _C // _L


def _sc_gather(src, dst, table_flat):
    mesh = plsc.VectorSubcoreMesh(core_axis_name="c", subcore_axis_name="s")

    @functools.partial(
        pl.kernel,
        mesh=mesh,
        out_type=jax.ShapeDtypeStruct((_B_PAD,), jnp.float32),
        scratch_types=[
            pltpu.VMEM((_C,), jnp.int32),
            pltpu.VMEM((_C,), jnp.int32),
            pltpu.VMEM((_C,), jnp.int32),
            pltpu.VMEM((_C,), jnp.float32),
            pltpu.SemaphoreType.DMA,
        ],
    )
    def k(src_hbm, dst_hbm, tab_hbm, out_hbm, src_v, dst_v, idx_v, val_v, sem):
        wid = lax.axis_index("s") * _NC + lax.axis_index("c")
        base = wid * _PW

        def chunk_body(g, carry):
            off = base + g * _C
            pltpu.sync_copy(src_hbm.at[pl.ds(off, _C)], src_v)
            pltpu.sync_copy(dst_hbm.at[pl.ds(off, _C)], dst_v)

            def flat_body(j, c2):
                s = src_v[pl.ds(j * _L, _L)]
                d = dst_v[pl.ds(j * _L, _L)]
                idx_v[pl.ds(j * _L, _L)] = s * _NUM_NODES + d
                return c2

            lax.fori_loop(0, _IN_ITERS, flat_body, 0)
            pltpu.async_copy(tab_hbm.at[idx_v], val_v, sem).wait()
            pltpu.sync_copy(val_v, out_hbm.at[pl.ds(off, _C)])
            return carry

        lax.fori_loop(0, _NCH, chunk_body, 0)

    return k(src, dst, table_flat)


def kernel(src, dst, distance_matrix):
    src = src.astype(jnp.int32)
    dst = dst.astype(jnp.int32)
    pad = _B_PAD - _B
    src_p = jnp.concatenate([src, src[:pad]])
    dst_p = jnp.concatenate([dst, dst[:pad]])
    flat = distance_matrix.reshape(-1)
    out = _sc_gather(src_p, dst_p, flat)
    return out[:_B]


# speedup vs baseline: 1.0834x; 1.0834x over previous
import functools

import jax
import jax.numpy as jnp
from jax import lax
from jax.experimental import pallas as pl
from jax.experimental.pallas import tpu as pltpu
from jax.experimental.pallas import tpu_sc as plsc

_NUM_NODES = 10000
_B = 1_000_000
_B_PAD = 1 << 20
_NC, _NS, _L = 2, 16, 16
_NW = _NC * _NS
_PW = _B_PAD // _NW
_C = 2048
_NCH = _PW // _C
_IN_ITERS = _C // _L


def _sc_gather(src, dst, table_flat):
    mesh = plsc.VectorSubcoreMesh(core_axis_name="c", subcore_axis_name="s")

    @functools.partial(
        pl.kernel,
        mesh=mesh,
        out_type=jax.ShapeDtypeStruct((_B_PAD,), jnp.float32),
        scratch_types=[
            pltpu.VMEM((2, _C), jnp.int32),
            pltpu.VMEM((2, _C), jnp.int32),
            pltpu.VMEM((_PW,), jnp.int32),
            pltpu.VMEM((_PW,), jnp.float32),
            pltpu.SemaphoreType.DMA,
            pltpu.SemaphoreType.DMA,
        ],
    )
    def k(src_hbm, dst_hbm, tab_hbm, out_hbm, src_v, dst_v, idx_v, val_v,
          sem_in, sem_g):
        wid = lax.axis_index("s") * _NC + lax.axis_index("c")
        base = wid * _PW

        def stage(g, buf):
            off = base + g * _C
            a = pltpu.async_copy(src_hbm.at[pl.ds(off, _C)], src_v.at[buf],
                                 sem_in)
            b = pltpu.async_copy(dst_hbm.at[pl.ds(off, _C)], dst_v.at[buf],
                                 sem_in)
            return a, b

        pending = stage(0, 0)
        gathers = []
        for g in range(_NCH):
            buf = g % 2
            for h in pending:
                h.wait()
            if g + 1 < _NCH:
                pending = stage(g + 1, (g + 1) % 2)

            def flat_body(j, _, buf=buf, g=g):
                s = src_v[buf, pl.ds(j * _L, _L)]
                d = dst_v[buf, pl.ds(j * _L, _L)]
                idx_v[pl.ds(g * _C + j * _L, _L)] = s * _NUM_NODES + d
                return 0

            lax.fori_loop(0, _IN_ITERS, flat_body, 0)
            gathers.append(
                pltpu.async_copy(
                    tab_hbm.at[idx_v.at[pl.ds(g * _C, _C)]],
                    val_v.at[pl.ds(g * _C, _C)],
                    sem_g,
                )
            )
        for h in gathers:
            h.wait()
        pltpu.sync_copy(val_v, out_hbm.at[pl.ds(base, _PW)])

    return k(src, dst, table_flat)


def kernel(src, dst, distance_matrix):
    src = src.astype(jnp.int32)
    dst = dst.astype(jnp.int32)
    pad = _B_PAD - _B
    src_p = jnp.concatenate([src, src[:pad]])
    dst_p = jnp.concatenate([dst, dst[:pad]])
    flat = distance_matrix.reshape(-1)
    out = _sc_gather(src_p, dst_p, flat)
    return out[:_B]


# speedup vs baseline: 1.0866x; 1.0030x over previous
import functools

import jax
import jax.numpy as jnp
from jax import lax
from jax.experimental import pallas as pl
from jax.experimental.pallas import tpu as pltpu
from jax.experimental.pallas import tpu_sc as plsc

_NUM_NODES = 10000
_B = 1_000_000
_B_PAD = 1 << 20
_NC, _NS, _L = 2, 16, 16
_NW = _NC * _NS
_PW = _B_PAD // _NW
_C = 4096
_NCH = _PW // _C


def _sc_gather(src, dst, table_flat):
    mesh = plsc.VectorSubcoreMesh(core_axis_name="c", subcore_axis_name="s")

    @functools.partial(
        pl.kernel,
        mesh=mesh,
        out_type=jax.ShapeDtypeStruct((_B_PAD,), jnp.float32),
        scratch_types=[
            pltpu.VMEM((2, _C), jnp.int32),
            pltpu.VMEM((2, _C), jnp.int32),
            pltpu.VMEM((_PW,), jnp.int32),
            pltpu.VMEM((_PW,), jnp.float32),
            pltpu.SemaphoreType.DMA,
            pltpu.SemaphoreType.DMA,
            pltpu.SemaphoreType.DMA,
        ],
    )
    def k(src_hbm, dst_hbm, tab_hbm, out_hbm, src_v, dst_v, idx_v, val_v,
          sem_in, sem_g, sem_wb):
        wid = lax.axis_index("s") * _NC + lax.axis_index("c")
        base = wid * _PW

        def stage(g, buf):
            off = base + g * _C
            a = pltpu.async_copy(src_hbm.at[pl.ds(off, _C)], src_v.at[buf],
                                 sem_in)
            b = pltpu.async_copy(dst_hbm.at[pl.ds(off, _C)], dst_v.at[buf],
                                 sem_in)
            return a, b

        pending = stage(0, 0)
        gathers = []
        for g in range(_NCH):
            buf = g % 2
            for h in pending:
                h.wait()
            if g + 1 < _NCH:
                pending = stage(g + 1, (g + 1) % 2)

            slot = g * _C

            @plsc.parallel_loop(0, _C, step=_L, unroll=8)
            def _(j, buf=buf, slot=slot):
                s = src_v[buf, pl.ds(j, _L)]
                d = dst_v[buf, pl.ds(j, _L)]
                idx_v[pl.ds(slot + j, _L)] = s * _NUM_NODES + d

            gathers.append(
                pltpu.async_copy(
                    tab_hbm.at[idx_v.at[pl.ds(slot, _C)]],
                    val_v.at[pl.ds(slot, _C)],
                    sem_g,
                )
            )
        writebacks = []
        for g in range(_NCH):
            gathers[g].wait()
            writebacks.append(
                pltpu.async_copy(val_v.at[pl.ds(g * _C, _C)],
                                 out_hbm.at[pl.ds(base + g * _C, _C)],
                                 sem_wb)
            )
        for h in writebacks:
            h.wait()

    return k(src, dst, table_flat)


def kernel(src, dst, distance_matrix):
    src = src.astype(jnp.int32)
    dst = dst.astype(jnp.int32)
    pad = _B_PAD - _B
    src_p = jnp.concatenate([src, src[:pad]])
    dst_p = jnp.concatenate([dst, dst[:pad]])
    flat = distance_matrix.reshape(-1)
    out = _sc_gather(src_p, dst_p, flat)
    return out[:_B]


# speedup vs baseline: 1.0866x; 1.0000x over previous
import functools

import jax
import jax.numpy as jnp
from jax import lax
from jax.experimental import pallas as pl
from jax.experimental.pallas import tpu as pltpu
from jax.experimental.pallas import tpu_sc as plsc

_NUM_NODES = 10000
_B = 1_000_000
_B_PAD = 1 << 20
_NC, _NS, _L = 2, 16, 16
_NW = _NC * _NS
_PW = _B_PAD // _NW
_C = 8192
_NCH = _PW // _C


def _sc_gather(src, dst, table_flat):
    mesh = plsc.VectorSubcoreMesh(core_axis_name="c", subcore_axis_name="s")

    @functools.partial(
        pl.kernel,
        mesh=mesh,
        out_type=jax.ShapeDtypeStruct((_B_PAD,), jnp.float32),
        scratch_types=[
            pltpu.VMEM((2, _C), jnp.int32),
            pltpu.VMEM((2, _C), jnp.int32),
            pltpu.VMEM((_PW,), jnp.int32),
            pltpu.VMEM((_PW,), jnp.float32),
            pltpu.SemaphoreType.DMA,
            pltpu.SemaphoreType.DMA,
            pltpu.SemaphoreType.DMA,
        ],
    )
    def k(src_hbm, dst_hbm, tab_hbm, out_hbm, src_v, dst_v, idx_v, val_v,
          sem_in, sem_g, sem_wb):
        wid = lax.axis_index("s") * _NC + lax.axis_index("c")
        base = wid * _PW

        def stage(g, buf):
            off = base + g * _C
            a = pltpu.async_copy(src_hbm.at[pl.ds(off, _C)], src_v.at[buf],
                                 sem_in)
            b = pltpu.async_copy(dst_hbm.at[pl.ds(off, _C)], dst_v.at[buf],
                                 sem_in)
            return a, b

        pending = stage(0, 0)
        gathers = []
        for g in range(_NCH):
            buf = g % 2
            for h in pending:
                h.wait()
            if g + 1 < _NCH:
                pending = stage(g + 1, (g + 1) % 2)

            slot = g * _C

            @plsc.parallel_loop(0, _C, step=_L, unroll=8)
            def _(j, buf=buf, slot=slot):
                s = src_v[buf, pl.ds(j, _L)]
                d = dst_v[buf, pl.ds(j, _L)]
                idx_v[pl.ds(slot + j, _L)] = s * _NUM_NODES + d

            gathers.append(
                pltpu.async_copy(
                    tab_hbm.at[idx_v.at[pl.ds(slot, _C)]],
                    val_v.at[pl.ds(slot, _C)],
                    sem_g,
                )
            )
        writebacks = []
        for g in range(_NCH):
            gathers[g].wait()
            writebacks.append(
                pltpu.async_copy(val_v.at[pl.ds(g * _C, _C)],
                                 out_hbm.at[pl.ds(base + g * _C, _C)],
                                 sem_wb)
            )
        for h in writebacks:
            h.wait()

    return k(src, dst, table_flat)


def kernel(src, dst, distance_matrix):
    src = src.astype(jnp.int32)
    dst = dst.astype(jnp.int32)
    pad = _B_PAD - _B
    src_p = jnp.concatenate([src, src[:pad]])
    dst_p = jnp.concatenate([dst, dst[:pad]])
    flat = distance_matrix.reshape(-1)
    out = _sc_gather(src_p, dst_p, flat)
    return out[:_B]
